# Initial kernel scaffold; baseline (speedup 1.0000x reference)
import jax
import jax.numpy as jnp
from jax.experimental import pallas as pl


def kernel(x, edge_attr, params, edge_index, batch):
    raise NotImplementedError("write your pallas kernel here")


# speedup vs baseline: 11.8522x; 11.8522x over previous
import functools
import math

import jax
import jax.numpy as jnp
from jax import lax
from jax.experimental import pallas as pl
from jax.experimental.pallas import tpu as pltpu
from jax.experimental.pallas import tpu_sc as plsc

N = 10000
E = 320000
C = 64
DE = 16
NG = 64

NC = 2
NS = 16
NW = NC * NS
EPT = E // NW
EC = 80
NCHUNK = EPT // EC
NPAD = 10240
ROWS_PER_TILE = NPAD // NS
ZR = 128
MW = 128

_BN_SCALE = 1.0 / math.sqrt(1.0 + 1e-5)


def _edge_body(kv_hbm, qqe_hbm, ea_hbm, src_hbm, dst_hbm, out_hbm,
               srcv, dstv, kvb, qb, eab, msgb, acc_sh, sem1, sem2):
    c = lax.axis_index("c")
    sid = lax.axis_index("s")
    wid = c * NS + sid

    def zrow(i, _):
        r = i // (MW // 16)
        col = (i % (MW // 16)) * 16
        msgb[r, pl.ds(col, 16)] = jnp.zeros((16,), jnp.float32)
        return 0
    lax.fori_loop(0, EC * (MW // 16), zrow, 0)
    def zcopy(i, _):
        pltpu.sync_copy(msgb, acc_sh.at[pl.ds(sid * ROWS_PER_TILE + i * EC, EC)])
        return 0
    lax.fori_loop(0, ROWS_PER_TILE // EC, zcopy, 0)
    plsc.subcore_barrier()

    def chunk_body(ci, _):
        off = wid * EPT + ci * EC
        pltpu.sync_copy(src_hbm.at[pl.ds(off, EC)], srcv)
        pltpu.sync_copy(dst_hbm.at[pl.ds(off, EC)], dstv)
        pltpu.sync_copy(ea_hbm.at[pl.ds(off, EC)], eab)
        cp1 = pltpu.async_copy(kv_hbm.at[srcv], kvb, sem1)
        cp2 = pltpu.async_copy(qqe_hbm.at[dstv], qb, sem2)
        cp1.wait()
        cp2.wait()

        def edge4_body(t, _):
            for u in range(4):
                e = t * 4 + u
                vacc = kvb[e, pl.ds(0, 16)] * qb[e, pl.ds(0, 16)]
                for gk in range(1, 4):
                    vacc = vacc + kvb[e, pl.ds(gk * 16, 16)] * qb[e, pl.ds(gk * 16, 16)]
                vacc = vacc + eab[e, pl.ds(0, 16)] * qb[e, pl.ds(64, 16)]
                alpha = jnp.sum(vacc) * 0.125
                ex = jnp.exp(jnp.full((16,), alpha, jnp.float32))
                for gk in range(4):
                    msgb[e, pl.ds(gk * 16, 16)] = kvb[e, pl.ds(64 + gk * 16, 16)] * ex
                msgb[e, pl.ds(64, 16)] = eab[e, pl.ds(0, 16)] * ex
                msgb[e, pl.ds(80, 16)] = ex
            return 0
        lax.fori_loop(0, EC // 4, edge4_body, 0)

        pltpu.sync_copy(msgb, acc_sh.at[dstv], add=True)
        return 0
    lax.fori_loop(0, NCHUNK, chunk_body, 0)

    plsc.subcore_barrier()
    pltpu.sync_copy(acc_sh.at[pl.ds(sid * ROWS_PER_TILE, ROWS_PER_TILE)],
                    out_hbm.at[pl.ds(c * NPAD + sid * ROWS_PER_TILE, ROWS_PER_TILE)])


_edge_call = functools.partial(
    pl.kernel,
    out_type=jax.ShapeDtypeStruct((2 * NPAD, MW), jnp.float32),
    mesh=plsc.VectorSubcoreMesh(core_axis_name="c", subcore_axis_name="s"),
    compiler_params=pltpu.CompilerParams(needs_layout_passes=False),
    scratch_types=[
        pltpu.VMEM((EC,), jnp.int32),
        pltpu.VMEM((EC,), jnp.int32),
        pltpu.VMEM((EC, 128), jnp.float32),
        pltpu.VMEM((EC, 128), jnp.float32),
        pltpu.VMEM((EC, DE), jnp.float32),
        pltpu.VMEM((EC, MW), jnp.float32),
        pltpu.VMEM_SHARED((NPAD, MW), jnp.float32),
        pltpu.SemaphoreType.DMA,
        pltpu.SemaphoreType.DMA,
    ],
)(_edge_body)


def _proj_body(h_ref, w_ref, b_ref, we_ref, kv_ref, qqe_ref, sx_ref):
    h = h_ref[...]
    hw = jnp.dot(h, w_ref[...].T, preferred_element_type=jnp.float32) + b_ref[...]
    q = hw[:, 0:64]
    kv_ref[...] = hw[:, 64:192]
    qe = jnp.dot(q, we_ref[...], preferred_element_type=jnp.float32)
    qqe_ref[...] = jnp.concatenate(
        [q, qe, jnp.zeros((q.shape[0], 48), jnp.float32)], axis=1)
    sx_ref[...] = hw[:, 192:256]


def _proj_call(h, wall, ball, we):
    din = h.shape[1]
    br = 2000
    grid = N // br
    return pl.pallas_call(
        _proj_body,
        grid=(grid,),
        in_specs=[
            pl.BlockSpec((br, din), lambda i: (i, 0)),
            pl.BlockSpec((256, din), lambda i: (0, 0)),
            pl.BlockSpec((1, 256), lambda i: (0, 0)),
            pl.BlockSpec((64, DE), lambda i: (0, 0)),
        ],
        out_specs=[
            pl.BlockSpec((br, 128), lambda i: (i, 0)),
            pl.BlockSpec((br, 128), lambda i: (i, 0)),
            pl.BlockSpec((br, 64), lambda i: (i, 0)),
        ],
        out_shape=[
            jax.ShapeDtypeStruct((N, 128), jnp.float32),
            jax.ShapeDtypeStruct((N, 128), jnp.float32),
            jax.ShapeDtypeStruct((N, 64), jnp.float32),
        ],
    )(h, wall, ball, we)


def _combine_body(acc_ref, sx_ref, p_ref, wt_ref, bt_ref, g_ref, bb_ref, h_ref):
    a = acc_ref[0] + acc_ref[1]
    num = jnp.dot(a, p_ref[...], preferred_element_type=jnp.float32)
    s = a[:, 80:81]
    out = num / (s + 1e-16) + sx_ref[...]
    hh = jnp.maximum(jnp.dot(out, wt_ref[...].T, preferred_element_type=jnp.float32)
                     + bt_ref[...], 0.0)
    h_ref[...] = hh * g_ref[...] + bb_ref[...]


def _combine_call(acc, sx, pmat, wt, bt, g, bb):
    br = 2000
    grid = N // br
    return pl.pallas_call(
        _combine_body,
        grid=(grid,),
        in_specs=[
            pl.BlockSpec((2, br, MW), lambda i: (0, i, 0)),
            pl.BlockSpec((br, 64), lambda i: (i, 0)),
            pl.BlockSpec((MW, 64), lambda i: (0, 0)),
            pl.BlockSpec((64, 64), lambda i: (0, 0)),
            pl.BlockSpec((1, 64), lambda i: (0, 0)),
            pl.BlockSpec((1, 64), lambda i: (0, 0)),
            pl.BlockSpec((1, 64), lambda i: (0, 0)),
        ],
        out_specs=pl.BlockSpec((br, 64), lambda i: (i, 0)),
        out_shape=jax.ShapeDtypeStruct((N, 64), jnp.float32),
    )(acc, sx, pmat, wt, bt, g, bb)


def _pool_body(h_ref, b_ref, w1_ref, b1_ref, w2_ref, b2_ref, w3_ref, b3_ref,
               pooled_ref, o_ref, acc_ref):
    i = pl.program_id(0)

    @pl.when(i == 0)
    def _():
        acc_ref[...] = jnp.zeros_like(acc_ref)

    bids = b_ref[0]
    gid = lax.broadcasted_iota(jnp.int32, (NG, 1000), 0)
    oh = (bids == gid).astype(jnp.float32)
    h = h_ref[...]
    haug = jnp.concatenate([h, jnp.ones((1000, 64), jnp.float32)], axis=1)
    acc_ref[...] += jnp.dot(oh, haug, preferred_element_type=jnp.float32)

    @pl.when(i == pl.num_programs(0) - 1)
    def _():
        acc = acc_ref[...]
        cnt = jnp.maximum(acc[:, 64:65], 1.0)
        pooled = acc[:, 0:64] / cnt
        pooled_ref[...] = pooled
        t = jnp.maximum(jnp.dot(pooled, w1_ref[...].T, preferred_element_type=jnp.float32)
                        + b1_ref[...], 0.0)
        t = jnp.maximum(jnp.dot(t, w2_ref[...].T, preferred_element_type=jnp.float32)
                        + b2_ref[...], 0.0)
        lg = jnp.dot(t, w3_ref[...].T, preferred_element_type=jnp.float32) + b3_ref[...]
        m = jnp.max(lg, axis=1, keepdims=True)
        e = jnp.exp(lg - m)
        o_ref[...] = e / jnp.sum(e, axis=1, keepdims=True)


def _pool_call(h, batch3, p1, p2, p3):
    br = 1000
    grid = N // br
    return pl.pallas_call(
        _pool_body,
        grid=(grid,),
        in_specs=[
            pl.BlockSpec((br, 64), lambda i: (i, 0)),
            pl.BlockSpec((1, 1, br), lambda i: (i, 0, 0)),
            pl.BlockSpec((64, 64), lambda i: (0, 0)),
            pl.BlockSpec((1, 64), lambda i: (0, 0)),
            pl.BlockSpec((32, 64), lambda i: (0, 0)),
            pl.BlockSpec((1, 32), lambda i: (0, 0)),
            pl.BlockSpec((2, 32), lambda i: (0, 0)),
            pl.BlockSpec((1, 2), lambda i: (0, 0)),
        ],
        out_specs=[
            pl.BlockSpec((NG, 64), lambda i: (0, 0)),
            pl.BlockSpec((NG, 2), lambda i: (0, 0)),
        ],
        out_shape=[
            jax.ShapeDtypeStruct((NG, 64), jnp.float32),
            jax.ShapeDtypeStruct((NG, 2), jnp.float32),
        ],
        scratch_shapes=[pltpu.VMEM((NG, 128), jnp.float32)],
    )(h, batch3, p1["W"], p1["b"].reshape(1, -1), p2["W"], p2["b"].reshape(1, -1),
      p3["W"], p3["b"].reshape(1, -1))


def _layer(h, edge_attr, srcs, dsts, cp, tp, bnp):
    wall = jnp.concatenate([cp["q"]["W"], cp["k"]["W"], cp["v"]["W"], cp["s"]["W"]], axis=0)
    ball = jnp.concatenate([cp["q"]["b"], cp["k"]["b"], cp["v"]["b"], cp["s"]["b"]]).reshape(1, 256)
    we = cp["e"]["W"]
    kv, qqe, sx = _proj_call(h, wall, ball, we)
    acc = _edge_call(kv, qqe, edge_attr, srcs, dsts)
    acc = acc.reshape(2, NPAD, MW)[:, :N, :]
    pmat = jnp.concatenate([jnp.eye(64, dtype=jnp.float32), we.T,
                            jnp.zeros((48, 64), jnp.float32)], axis=0)
    g_eff = (bnp["g"] * _BN_SCALE).reshape(1, 64)
    return _combine_call(acc, sx, pmat, tp["W"], tp["b"].reshape(1, 64),
                         g_eff, bnp["b"].reshape(1, 64))


def kernel(x, edge_attr, params, edge_index, batch):
    srcs = edge_index[0]
    dsts = edge_index[1]
    h = x
    convs = [params["conv1"]] + list(params["convs"])
    transfs = [params["transf1"]] + list(params["transfs"])
    bns = [params["bn1"]] + list(params["bns"])
    for li in range(4):
        h = _layer(h, edge_attr, srcs, dsts, convs[li], transfs[li], bns[li])
    batch3 = batch.reshape(10, 1, 1000)
    pooled, o = _pool_call(h, batch3, params["lin1"], params["lin2"], params["lin3"])
    return pooled, o
